# speedup vs baseline: 1.4843x; 1.4843x over previous
import functools

import jax
import jax.numpy as jnp
from jax import lax
from jax.experimental import pallas as pl
from jax.experimental.pallas import tpu as pltpu
from jax.experimental.pallas import tpu_sc as plsc

VOCAB = 100000
HIDDEN = 2048
B = 16384

_NC = 2
_NS = 16
_NW = _NC * _NS
_BPW = B // _NW
_C = 16
_NCH = _BPW // _C

_mesh = plsc.VectorSubcoreMesh(core_axis_name="c", subcore_axis_name="s")


@functools.partial(
    pl.kernel,
    mesh=_mesh,
    out_type=jax.ShapeDtypeStruct((B, HIDDEN), jnp.float32),
    scratch_types=[
        pltpu.VMEM((_BPW,), jnp.int32),
        pltpu.VMEM((_C, HIDDEN), jnp.float32),
        pltpu.SemaphoreType.DMA,
    ],
)
def _emb_lookup(ids_hbm, table_hbm, out_hbm, idx_v, rows_v, sem):
    wid = lax.axis_index("s") * _NC + lax.axis_index("c")
    base = wid * _BPW
    pltpu.sync_copy(ids_hbm.at[pl.ds(base, _BPW)], idx_v)

    def body(g, carry):
        c0 = pl.multiple_of(g * _C, 8)
        pltpu.async_copy(table_hbm.at[idx_v.at[pl.ds(c0, _C)]], rows_v, sem).wait()
        pltpu.sync_copy(rows_v, out_hbm.at[pl.ds(base + c0, _C)])
        return carry

    lax.fori_loop(0, _NCH, body, 0)


def kernel(input_ids, word_embeddings):
    ids = input_ids.reshape(-1).astype(jnp.int32)
    out = _emb_lookup(ids, word_embeddings)
    return out.reshape(input_ids.shape + (word_embeddings.shape[1],))


# speedup vs baseline: 1.7800x; 1.1992x over previous
import functools

import jax
import jax.numpy as jnp
from jax import lax
from jax.experimental import pallas as pl
from jax.experimental.pallas import tpu as pltpu
from jax.experimental.pallas import tpu_sc as plsc

VOCAB = 100000
HIDDEN = 2048
B = 16384

_NC = 2
_NS = 16
_NW = _NC * _NS
_BPW = B // _NW
_C = 16
_NCH = _BPW // _C

_mesh = plsc.VectorSubcoreMesh(core_axis_name="c", subcore_axis_name="s")


@functools.partial(
    pl.kernel,
    mesh=_mesh,
    out_type=jax.ShapeDtypeStruct((B, HIDDEN), jnp.float32),
    scratch_types=[
        pltpu.VMEM((_BPW,), jnp.int32),
        pltpu.VMEM((_C, HIDDEN), jnp.float32),
        pltpu.VMEM((_C, HIDDEN), jnp.float32),
        pltpu.SemaphoreType.DMA,
        pltpu.SemaphoreType.DMA,
        pltpu.SemaphoreType.DMA,
        pltpu.SemaphoreType.DMA,
    ],
)
def _emb_lookup(ids_hbm, table_hbm, out_hbm, idx_v, rows0, rows1,
                gsem0, gsem1, osem0, osem1):
    wid = lax.axis_index("s") * _NC + lax.axis_index("c")
    base = wid * _BPW
    pltpu.sync_copy(ids_hbm.at[pl.ds(base, _BPW)], idx_v)

    bufs = (rows0, rows1)
    gsems = (gsem0, gsem1)
    osems = (osem0, osem1)

    def gather_start(g, slot):
        c0 = pl.multiple_of(g * _C, 8)
        pltpu.async_copy(table_hbm.at[idx_v.at[pl.ds(c0, _C)]],
                         bufs[slot], gsems[slot])

    def gather_wait(g, slot):
        c0 = pl.multiple_of(g * _C, 8)
        pltpu.make_async_copy(table_hbm.at[idx_v.at[pl.ds(c0, _C)]],
                              bufs[slot], gsems[slot]).wait()

    def put_start(g, slot):
        c0 = pl.multiple_of(g * _C, 8)
        pltpu.async_copy(bufs[slot], out_hbm.at[pl.ds(base + c0, _C)],
                         osems[slot])

    def put_wait(g, slot):
        c0 = pl.multiple_of(g * _C, 8)
        pltpu.make_async_copy(bufs[slot], out_hbm.at[pl.ds(base + c0, _C)],
                              osems[slot]).wait()

    gather_start(0, 0)
    gather_start(1, 1)

    def step(g, slot):
        gather_wait(g, slot)
        put_start(g, slot)
        put_wait(g, slot)
        gather_start(g + 2, slot)

    def pair(p, carry):
        g0 = p * 2
        step(g0, 0)
        step(g0 + 1, 1)
        return carry

    lax.fori_loop(0, (_NCH - 2) // 2, pair, 0)
    for g, slot in ((_NCH - 2, 0), (_NCH - 1, 1)):
        gather_wait(g, slot)
        put_start(g, slot)
        put_wait(g, slot)


def kernel(input_ids, word_embeddings):
    ids = input_ids.reshape(-1).astype(jnp.int32)
    out = _emb_lookup(ids, word_embeddings)
    return out.reshape(input_ids.shape + (word_embeddings.shape[1],))
